# Initial kernel scaffold; baseline (speedup 1.0000x reference)
import functools

import jax
import jax.numpy as jnp
from jax import lax
from jax.experimental import pallas as pl
from jax.experimental.pallas import tpu as pltpu
from jax.experimental.pallas import tpu_sc as plsc

N_NODES = 10000
N_EDGES = 320000
D = 128

NC = 2
NS = 16
L = 16
NW = NC * NS
NPW = N_EDGES // NW
C = 80
NCHUNK = NPW // C
NGRP = C // L


def _body(x_hbm, src_hbm, dst_hbm, neg_hbm, outp_hbm, outn_hbm,
          idx_s, idx_d, idx_n,
          sA, dA, nA, sB, dB, nB,
          op_v, on_v, semA, semB):
  wid = lax.axis_index("s") * NC + lax.axis_index("c")
  base = pl.multiple_of(wid * NPW, 8)

  pltpu.sync_copy(src_hbm.at[pl.ds(base, NPW)], idx_s)
  pltpu.sync_copy(dst_hbm.at[pl.ds(base, NPW)], idx_d)
  pltpu.sync_copy(neg_hbm.at[pl.ds(base, NPW)], idx_n)

  lane = lax.iota(jnp.int32, L)

  def start_gather(g, s_buf, d_buf, n_buf, sem):
    off = pl.multiple_of(g * C, 8)
    pltpu.make_async_copy(
        x_hbm.at[idx_s.at[pl.ds(off, C)]], s_buf, sem).start()
    pltpu.make_async_copy(
        x_hbm.at[idx_d.at[pl.ds(off, C)]], d_buf, sem).start()
    pltpu.make_async_copy(
        x_hbm.at[idx_n.at[pl.ds(off, C)]], n_buf, sem).start()

  def wait_gather(s_buf, d_buf, n_buf, sem):
    pltpu.make_async_copy(
        x_hbm.at[idx_s.at[pl.ds(0, C)]], s_buf, sem).wait()
    pltpu.make_async_copy(
        x_hbm.at[idx_d.at[pl.ds(0, C)]], d_buf, sem).wait()
    pltpu.make_async_copy(
        x_hbm.at[idx_n.at[pl.ds(0, C)]], n_buf, sem).wait()

  def compute(g, s_buf, d_buf, n_buf):
    coff = g * C

    def grp_body(k, carry):
      eg = k * L
      ovp = jnp.zeros((L,), jnp.float32)
      ovn = jnp.zeros((L,), jnp.float32)
      for i in range(L):
        e = eg + i
        s0 = s_buf[e, pl.ds(0, L)]
        accp = s0 * d_buf[e, pl.ds(0, L)]
        accn = s0 * n_buf[e, pl.ds(0, L)]
        for j in range(1, D // L):
          sj = s_buf[e, pl.ds(j * L, L)]
          accp = accp + sj * d_buf[e, pl.ds(j * L, L)]
          accn = accn + sj * n_buf[e, pl.ds(j * L, L)]
        rp = jnp.sum(accp)
        rn = jnp.sum(accn)
        ovp = jnp.where(lane == i, rp, ovp)
        ovn = jnp.where(lane == i, rn, ovn)
      o = pl.multiple_of(coff + eg, 8)
      op_v[pl.ds(o, L)] = ovp
      on_v[pl.ds(o, L)] = ovn
      return carry

    lax.fori_loop(0, NGRP, grp_body, 0)

  start_gather(0, sA, dA, nA, semA)

  def pipe(it, carry):
    ga = 2 * it
    start_gather(ga + 1, sB, dB, nB, semB)
    wait_gather(sA, dA, nA, semA)
    compute(ga, sA, dA, nA)
    start_gather(ga + 2, sA, dA, nA, semA)
    wait_gather(sB, dB, nB, semB)
    compute(ga + 1, sB, dB, nB)
    return carry

  lax.fori_loop(0, (NCHUNK - 1) // 2, pipe, 0)
  wait_gather(sA, dA, nA, semA)
  compute(NCHUNK - 1, sA, dA, nA)

  pltpu.sync_copy(op_v, outp_hbm.at[pl.ds(base, NPW)])
  pltpu.sync_copy(on_v, outn_hbm.at[pl.ds(base, NPW)])


@jax.jit
def _dot_decoder(x, src, dst, neg):
  mesh = plsc.VectorSubcoreMesh(core_axis_name="c", subcore_axis_name="s")
  run = pl.kernel(
      _body,
      out_type=(
          jax.ShapeDtypeStruct((N_EDGES,), jnp.float32),
          jax.ShapeDtypeStruct((N_EDGES,), jnp.float32),
      ),
      mesh=mesh,
      scratch_types=[
          pltpu.VMEM((NPW,), jnp.int32),
          pltpu.VMEM((NPW,), jnp.int32),
          pltpu.VMEM((NPW,), jnp.int32),
          pltpu.VMEM((C, D), jnp.float32),
          pltpu.VMEM((C, D), jnp.float32),
          pltpu.VMEM((C, D), jnp.float32),
          pltpu.VMEM((C, D), jnp.float32),
          pltpu.VMEM((C, D), jnp.float32),
          pltpu.VMEM((C, D), jnp.float32),
          pltpu.VMEM((NPW,), jnp.float32),
          pltpu.VMEM((NPW,), jnp.float32),
          pltpu.SemaphoreType.DMA,
          pltpu.SemaphoreType.DMA,
      ],
      name="dot_decoder_sc",
  )
  return run(x, src, dst, neg)


def kernel(x, edge_index, neg_dst):
  src = edge_index[0].astype(jnp.int32)
  dst = edge_index[1].astype(jnp.int32)
  neg = neg_dst.astype(jnp.int32)
  return _dot_decoder(x, src, dst, neg)


# speedup vs baseline: 1.2630x; 1.2630x over previous
import functools

import jax
import jax.numpy as jnp
from jax import lax
from jax.experimental import pallas as pl
from jax.experimental.pallas import tpu as pltpu
from jax.experimental.pallas import tpu_sc as plsc

N_NODES = 10000
N_EDGES = 320000
D = 128

NC = 2
NS = 16
L = 16
NW = NC * NS
NPW = N_EDGES // NW
C = 80
NCHUNK = NPW // C
NGRP = C // L


def _body(x_hbm, src_hbm, dst_hbm, neg_hbm, outp_hbm, outn_hbm,
          idx_s, idx_d, idx_n,
          sA, dA, nA, sB, dB, nB,
          op_v, on_v, semA, semB):
  wid = lax.axis_index("s") * NC + lax.axis_index("c")
  base = pl.multiple_of(wid * NPW, 8)

  pltpu.sync_copy(src_hbm.at[pl.ds(base, NPW)], idx_s)
  pltpu.sync_copy(dst_hbm.at[pl.ds(base, NPW)], idx_d)
  pltpu.sync_copy(neg_hbm.at[pl.ds(base, NPW)], idx_n)

  lane = lax.iota(jnp.int32, L)

  def start_gather(g, s_buf, d_buf, n_buf, sem):
    off = pl.multiple_of(g * C, 8)
    pltpu.make_async_copy(
        x_hbm.at[idx_s.at[pl.ds(off, C)]], s_buf, sem).start()
    pltpu.make_async_copy(
        x_hbm.at[idx_d.at[pl.ds(off, C)]], d_buf, sem).start()
    pltpu.make_async_copy(
        x_hbm.at[idx_n.at[pl.ds(off, C)]], n_buf, sem).start()

  def wait_gather(s_buf, d_buf, n_buf, sem):
    pltpu.make_async_copy(
        x_hbm.at[idx_s.at[pl.ds(0, C)]], s_buf, sem).wait()
    pltpu.make_async_copy(
        x_hbm.at[idx_d.at[pl.ds(0, C)]], d_buf, sem).wait()
    pltpu.make_async_copy(
        x_hbm.at[idx_n.at[pl.ds(0, C)]], n_buf, sem).wait()

  def compute(g, s_buf, d_buf, n_buf):
    coff = g * C

    def grp_body(k, carry):
      rows = k * L + lane
      sv = plsc.load_gather(s_buf, [rows, jnp.full((L,), 0, jnp.int32)])
      accp = sv * plsc.load_gather(d_buf, [rows, jnp.full((L,), 0, jnp.int32)])
      accn = sv * plsc.load_gather(n_buf, [rows, jnp.full((L,), 0, jnp.int32)])
      for d_ in range(1, D):
        col = jnp.full((L,), d_, jnp.int32)
        sv = plsc.load_gather(s_buf, [rows, col])
        accp = accp + sv * plsc.load_gather(d_buf, [rows, col])
        accn = accn + sv * plsc.load_gather(n_buf, [rows, col])
      o = pl.multiple_of(coff + k * L, 8)
      op_v[pl.ds(o, L)] = accp
      on_v[pl.ds(o, L)] = accn
      return carry

    lax.fori_loop(0, NGRP, grp_body, 0)

  start_gather(0, sA, dA, nA, semA)

  def pipe(it, carry):
    ga = 2 * it
    start_gather(ga + 1, sB, dB, nB, semB)
    wait_gather(sA, dA, nA, semA)
    compute(ga, sA, dA, nA)
    start_gather(ga + 2, sA, dA, nA, semA)
    wait_gather(sB, dB, nB, semB)
    compute(ga + 1, sB, dB, nB)
    return carry

  lax.fori_loop(0, (NCHUNK - 1) // 2, pipe, 0)
  wait_gather(sA, dA, nA, semA)
  compute(NCHUNK - 1, sA, dA, nA)

  pltpu.sync_copy(op_v, outp_hbm.at[pl.ds(base, NPW)])
  pltpu.sync_copy(on_v, outn_hbm.at[pl.ds(base, NPW)])


@jax.jit
def _dot_decoder(x, src, dst, neg):
  mesh = plsc.VectorSubcoreMesh(core_axis_name="c", subcore_axis_name="s")
  run = pl.kernel(
      _body,
      out_type=(
          jax.ShapeDtypeStruct((N_EDGES,), jnp.float32),
          jax.ShapeDtypeStruct((N_EDGES,), jnp.float32),
      ),
      mesh=mesh,
      scratch_types=[
          pltpu.VMEM((NPW,), jnp.int32),
          pltpu.VMEM((NPW,), jnp.int32),
          pltpu.VMEM((NPW,), jnp.int32),
          pltpu.VMEM((C, D), jnp.float32),
          pltpu.VMEM((C, D), jnp.float32),
          pltpu.VMEM((C, D), jnp.float32),
          pltpu.VMEM((C, D), jnp.float32),
          pltpu.VMEM((C, D), jnp.float32),
          pltpu.VMEM((C, D), jnp.float32),
          pltpu.VMEM((NPW,), jnp.float32),
          pltpu.VMEM((NPW,), jnp.float32),
          pltpu.SemaphoreType.DMA,
          pltpu.SemaphoreType.DMA,
      ],
      compiler_params=pltpu.CompilerParams(needs_layout_passes=False),
      name="dot_decoder_sc",
  )
  return run(x, src, dst, neg)


def kernel(x, edge_index, neg_dst):
  src = edge_index[0].astype(jnp.int32)
  dst = edge_index[1].astype(jnp.int32)
  neg = neg_dst.astype(jnp.int32)
  return _dot_decoder(x, src, dst, neg)


# speedup vs baseline: 4.3544x; 3.4477x over previous
import functools

import jax
import jax.numpy as jnp
from jax import lax
from jax.experimental import pallas as pl
from jax.experimental.pallas import tpu as pltpu
from jax.experimental.pallas import tpu_sc as plsc

N_NODES = 10000
N_EDGES = 320000
D = 128

NC = 2
NS = 16
L = 16
NW = NC * NS
NPW = N_EDGES // NW
C = 80
NCHUNK = NPW // C
NGRP = C // L


def _body(x_hbm, src_hbm, dst_hbm, neg_hbm, outp_hbm, outn_hbm,
          idx_s, idx_d, idx_n,
          sA, dA, nA, sB, dB, nB,
          op_v, on_v, semA, semB):
  wid = lax.axis_index("s") * NC + lax.axis_index("c")
  base = pl.multiple_of(wid * NPW, 8)

  pltpu.sync_copy(src_hbm.at[pl.ds(base, NPW)], idx_s)
  pltpu.sync_copy(dst_hbm.at[pl.ds(base, NPW)], idx_d)
  pltpu.sync_copy(neg_hbm.at[pl.ds(base, NPW)], idx_n)

  lane = lax.iota(jnp.int32, L)

  def start_gather(g, s_buf, d_buf, n_buf, sem):
    off = pl.multiple_of(g * C, 8)
    pltpu.make_async_copy(
        x_hbm.at[idx_s.at[pl.ds(off, C)]], s_buf, sem).start()
    pltpu.make_async_copy(
        x_hbm.at[idx_d.at[pl.ds(off, C)]], d_buf, sem).start()
    pltpu.make_async_copy(
        x_hbm.at[idx_n.at[pl.ds(off, C)]], n_buf, sem).start()

  def wait_gather(s_buf, d_buf, n_buf, sem):
    pltpu.make_async_copy(
        x_hbm.at[idx_s.at[pl.ds(0, C)]], s_buf, sem).wait()
    pltpu.make_async_copy(
        x_hbm.at[idx_d.at[pl.ds(0, C)]], d_buf, sem).wait()
    pltpu.make_async_copy(
        x_hbm.at[idx_n.at[pl.ds(0, C)]], n_buf, sem).wait()

  def compute(g, s_buf, d_buf, n_buf):
    coff = g * C
    zero = jnp.zeros((L,), jnp.float32)

    def zero_body(k, carry):
      o = coff + k * L
      op_v[pl.ds(o, L)] = zero
      on_v[pl.ds(o, L)] = zero
      return carry

    lax.fori_loop(0, NGRP, zero_body, 0)

    def edge_pair(i, carry):
      for u in range(2):
        e = i * 2 + u
        sv0 = s_buf[e, pl.ds(0, L)]
        accp = sv0 * d_buf[e, pl.ds(0, L)]
        accn = sv0 * n_buf[e, pl.ds(0, L)]
        for j in range(1, D // L):
          o = j * L
          sv = s_buf[e, pl.ds(o, L)]
          accp = accp + sv * d_buf[e, pl.ds(o, L)]
          accn = accn + sv * n_buf[e, pl.ds(o, L)]
        eidx = jnp.full((L,), coff + e, jnp.int32)
        plsc.addupdate_scatter(op_v, [eidx], accp)
        plsc.addupdate_scatter(on_v, [eidx], accn)
      return carry

    lax.fori_loop(0, C // 2, edge_pair, 0)

  start_gather(0, sA, dA, nA, semA)

  def pipe(it, carry):
    ga = 2 * it
    start_gather(ga + 1, sB, dB, nB, semB)
    wait_gather(sA, dA, nA, semA)
    compute(ga, sA, dA, nA)
    start_gather(ga + 2, sA, dA, nA, semA)
    wait_gather(sB, dB, nB, semB)
    compute(ga + 1, sB, dB, nB)
    return carry

  lax.fori_loop(0, (NCHUNK - 1) // 2, pipe, 0)
  wait_gather(sA, dA, nA, semA)
  compute(NCHUNK - 1, sA, dA, nA)

  pltpu.sync_copy(op_v, outp_hbm.at[pl.ds(base, NPW)])
  pltpu.sync_copy(on_v, outn_hbm.at[pl.ds(base, NPW)])


@jax.jit
def _dot_decoder(x, src, dst, neg):
  mesh = plsc.VectorSubcoreMesh(core_axis_name="c", subcore_axis_name="s")
  run = pl.kernel(
      _body,
      out_type=(
          jax.ShapeDtypeStruct((N_EDGES,), jnp.float32),
          jax.ShapeDtypeStruct((N_EDGES,), jnp.float32),
      ),
      mesh=mesh,
      scratch_types=[
          pltpu.VMEM((NPW,), jnp.int32),
          pltpu.VMEM((NPW,), jnp.int32),
          pltpu.VMEM((NPW,), jnp.int32),
          pltpu.VMEM((C, D), jnp.float32),
          pltpu.VMEM((C, D), jnp.float32),
          pltpu.VMEM((C, D), jnp.float32),
          pltpu.VMEM((C, D), jnp.float32),
          pltpu.VMEM((C, D), jnp.float32),
          pltpu.VMEM((C, D), jnp.float32),
          pltpu.VMEM((NPW,), jnp.float32),
          pltpu.VMEM((NPW,), jnp.float32),
          pltpu.SemaphoreType.DMA,
          pltpu.SemaphoreType.DMA,
      ],
      compiler_params=pltpu.CompilerParams(needs_layout_passes=False),
      name="dot_decoder_sc",
  )
  return run(x, src, dst, neg)


def kernel(x, edge_index, neg_dst):
  src = edge_index[0].astype(jnp.int32)
  dst = edge_index[1].astype(jnp.int32)
  neg = neg_dst.astype(jnp.int32)
  return _dot_decoder(x, src, dst, neg)


# speedup vs baseline: 6.2858x; 1.4436x over previous
import functools

import jax
import jax.numpy as jnp
from jax import lax
from jax.experimental import pallas as pl
from jax.experimental.pallas import tpu as pltpu
from jax.experimental.pallas import tpu_sc as plsc

N_NODES = 10000
N_EDGES = 320000
D = 128

NC = 2
NS = 16
L = 16
NW = NC * NS
NPW = N_EDGES // NW
C = 80
NCHUNK = NPW // C
NGRP = C // L


def _body(x_hbm, src_hbm, dst_hbm, neg_hbm, outp_hbm, outn_hbm,
          idx_s, idx_d, idx_n,
          sA, dA, nA, sB, dB, nB,
          op_v, on_v, scr_p, scr_n, semA, semB):
  wid = lax.axis_index("s") * NC + lax.axis_index("c")
  base = pl.multiple_of(wid * NPW, 8)

  pltpu.sync_copy(src_hbm.at[pl.ds(base, NPW)], idx_s)
  pltpu.sync_copy(dst_hbm.at[pl.ds(base, NPW)], idx_d)
  pltpu.sync_copy(neg_hbm.at[pl.ds(base, NPW)], idx_n)

  lane = lax.iota(jnp.int32, L)

  def start_gather(g, s_buf, d_buf, n_buf, sem):
    off = pl.multiple_of(g * C, 8)
    pltpu.make_async_copy(
        x_hbm.at[idx_s.at[pl.ds(off, C)]], s_buf, sem).start()
    pltpu.make_async_copy(
        x_hbm.at[idx_d.at[pl.ds(off, C)]], d_buf, sem).start()
    pltpu.make_async_copy(
        x_hbm.at[idx_n.at[pl.ds(off, C)]], n_buf, sem).start()

  def wait_gather(s_buf, d_buf, n_buf, sem):
    pltpu.make_async_copy(
        x_hbm.at[idx_s.at[pl.ds(0, C)]], s_buf, sem).wait()
    pltpu.make_async_copy(
        x_hbm.at[idx_d.at[pl.ds(0, C)]], d_buf, sem).wait()
    pltpu.make_async_copy(
        x_hbm.at[idx_n.at[pl.ds(0, C)]], n_buf, sem).wait()

  def compute(g, s_buf, d_buf, n_buf):
    coff = g * C

    def grp_body(k, carry):
      base_e = k * L
      for ee in range(L):
        e = base_e + ee
        sv0 = s_buf[e, pl.ds(0, L)]
        accp = sv0 * d_buf[e, pl.ds(0, L)]
        accn = sv0 * n_buf[e, pl.ds(0, L)]
        for j in range(1, D // L):
          o = j * L
          sv = s_buf[e, pl.ds(o, L)]
          accp = accp + sv * d_buf[e, pl.ds(o, L)]
          accn = accn + sv * n_buf[e, pl.ds(o, L)]
        scr_p[ee, pl.ds(0, L)] = accp
        scr_n[ee, pl.ds(0, L)] = accn
      col0 = jnp.full((L,), 0, jnp.int32)
      rp = plsc.load_gather(scr_p, [lane, col0])
      rn = plsc.load_gather(scr_n, [lane, col0])
      for c in range(1, L):
        colc = jnp.full((L,), c, jnp.int32)
        rp = rp + plsc.load_gather(scr_p, [lane, colc])
        rn = rn + plsc.load_gather(scr_n, [lane, colc])
      o = pl.multiple_of(coff + base_e, 8)
      op_v[pl.ds(o, L)] = rp
      on_v[pl.ds(o, L)] = rn
      return carry

    lax.fori_loop(0, NGRP, grp_body, 0)

  start_gather(0, sA, dA, nA, semA)

  def pipe(it, carry):
    ga = 2 * it
    start_gather(ga + 1, sB, dB, nB, semB)
    wait_gather(sA, dA, nA, semA)
    compute(ga, sA, dA, nA)
    start_gather(ga + 2, sA, dA, nA, semA)
    wait_gather(sB, dB, nB, semB)
    compute(ga + 1, sB, dB, nB)
    return carry

  lax.fori_loop(0, (NCHUNK - 1) // 2, pipe, 0)
  wait_gather(sA, dA, nA, semA)
  compute(NCHUNK - 1, sA, dA, nA)

  pltpu.sync_copy(op_v, outp_hbm.at[pl.ds(base, NPW)])
  pltpu.sync_copy(on_v, outn_hbm.at[pl.ds(base, NPW)])


@jax.jit
def _dot_decoder(x, src, dst, neg):
  mesh = plsc.VectorSubcoreMesh(core_axis_name="c", subcore_axis_name="s")
  run = pl.kernel(
      _body,
      out_type=(
          jax.ShapeDtypeStruct((N_EDGES,), jnp.float32),
          jax.ShapeDtypeStruct((N_EDGES,), jnp.float32),
      ),
      mesh=mesh,
      scratch_types=[
          pltpu.VMEM((NPW,), jnp.int32),
          pltpu.VMEM((NPW,), jnp.int32),
          pltpu.VMEM((NPW,), jnp.int32),
          pltpu.VMEM((C, D), jnp.float32),
          pltpu.VMEM((C, D), jnp.float32),
          pltpu.VMEM((C, D), jnp.float32),
          pltpu.VMEM((C, D), jnp.float32),
          pltpu.VMEM((C, D), jnp.float32),
          pltpu.VMEM((C, D), jnp.float32),
          pltpu.VMEM((NPW,), jnp.float32),
          pltpu.VMEM((NPW,), jnp.float32),
          pltpu.VMEM((L, L + 1), jnp.float32),
          pltpu.VMEM((L, L + 1), jnp.float32),
          pltpu.SemaphoreType.DMA,
          pltpu.SemaphoreType.DMA,
      ],
      compiler_params=pltpu.CompilerParams(needs_layout_passes=False),
      name="dot_decoder_sc",
  )
  return run(x, src, dst, neg)


def kernel(x, edge_index, neg_dst):
  src = edge_index[0].astype(jnp.int32)
  dst = edge_index[1].astype(jnp.int32)
  neg = neg_dst.astype(jnp.int32)
  return _dot_decoder(x, src, dst, neg)
